# baseline (speedup 1.0000x reference)
import functools

import jax
import jax.numpy as jnp
from jax import lax
from jax.experimental import pallas as pl
from jax.experimental.pallas import tpu as pltpu
from jax.experimental.pallas import tpu_sc as plsc

D = 300
L = 50
NC, NS = 2, 16
NW = NC * NS
NCHUNK = 19
GPR = 20
RPS = 2 * L
RPAD = 112
NG = GPR
V16 = 100000 * D // 16


def _sc_body(idx_hbm, emb_hbm, wck_hbm, out_hbm, idx_v, wck_v, out_v,
             ent_a, ent_b, dst_a, dst_b, sem_a, sem_b):
    wid = lax.axis_index("c") * NS + lax.axis_index("s")
    iters = idx_v.shape[0]
    row0 = wid * iters

    pltpu.sync_copy(idx_hbm.at[pl.ds(row0, iters)], idx_v)
    pltpu.sync_copy(wck_hbm, wck_v)

    iota = lax.iota(jnp.int32, 16)

    def build(it, ent):
        for m in range(7):
            rcl = jnp.minimum(iota + (16 * m), RPS - 1)
            v = plsc.load_gather(idx_v, [jnp.full((16,), it, jnp.int32),
                                         rcl])
            ek = lax.shift_right_logical(v * D, 4)
            for k in range(GPR):
                ent[k, pl.ds(16 * m, 16)] = jnp.minimum(ek, V16 - 1)
                ek = ek + 1

    def start(ent, dst, sem):
        for k in range(NG):
            pltpu.async_copy(emb_hbm.at[ent.at[k]],
                             dst.at[pl.ds(RPAD * k, RPAD)], sem)

    def wait(ent, dst, sem):
        for k in range(NG):
            pltpu.make_async_copy(emb_hbm.at[ent.at[k]],
                                  dst.at[pl.ds(RPAD * k, RPAD)], sem).wait()

    def compute(dst, it):
        def row_vecs(rg):
            v = plsc.load_gather(idx_v, [jnp.full((16,), it, jnp.int32),
                                         jnp.full((16,), rg, jnp.int32)])
            t = jnp.bitwise_and(v * D, 15) + iota
            rb0 = lax.shift_right_logical(t, 4) * RPAD + rg
            cols = jnp.bitwise_and(t, 15)
            return rb0, cols

        def chunk_loads(rg):
            rb0, cols = row_vecs(rg)
            return tuple(plsc.load_gather(dst, [rb0 + (RPAD * c), cols])
                         for c in range(NCHUNK))

        for half in range(2):
            base = half * L

            def body(r, accs):
                return tuple(jnp.maximum(a, x)
                             for a, x in zip(accs, chunk_loads(base + r)))

            accs = lax.fori_loop(1, L, body, chunk_loads(base))
            psum = jnp.zeros((16,), jnp.float32)
            for c in range(NCHUNK):
                psum = psum + accs[c] * wck_v[c, :]
            out_v[2 * it + half, :] = psum

    build(0, ent_a)
    start(ent_a, dst_a, sem_a)

    def step2(k2, _):
        it0 = 2 * k2
        it1 = 2 * k2 + 1
        build(it1, ent_b)
        start(ent_b, dst_b, sem_b)
        wait(ent_a, dst_a, sem_a)
        compute(dst_a, it0)
        it2 = jnp.minimum(it0 + 2, iters - 1)
        build(it2, ent_a)
        start(ent_a, dst_a, sem_a)
        wait(ent_b, dst_b, sem_b)
        compute(dst_b, it1)
        return _

    lax.fori_loop(0, iters // 2, step2, 0)
    wait(ent_a, dst_a, sem_a)
    pltpu.sync_copy(out_v, out_hbm.at[pl.ds(row0 * 2, 2 * iters)])


def _make_sc_call(bags, iters):
    mesh = plsc.VectorSubcoreMesh(
        core_axis_name="c", subcore_axis_name="s",
        num_cores=NC, num_subcores=NS)
    return functools.partial(
        pl.kernel, mesh=mesh,
        compiler_params=pltpu.CompilerParams(
            use_tc_tiling_on_sc=False, needs_layout_passes=False),
        out_type=jax.ShapeDtypeStruct((bags, 16), jnp.float32),
        scratch_types=[
            pltpu.VMEM((iters, RPS), jnp.int32),
            pltpu.VMEM((NCHUNK, 16), jnp.float32),
            pltpu.VMEM((2 * iters, 16), jnp.float32),
            pltpu.VMEM((GPR, RPAD), jnp.int32),
            pltpu.VMEM((GPR, RPAD), jnp.int32),
            pltpu.VMEM((GPR * RPAD, 16), jnp.float32),
            pltpu.VMEM((GPR * RPAD, 16), jnp.float32),
            pltpu.SemaphoreType.DMA,
            pltpu.SemaphoreType.DMA,
        ])(_sc_body)


def _tc_body(part_ref, lab_ref, b_ref, logits_ref, loss_ref):
    part = part_ref[:]
    s = jnp.sum(part, axis=1, keepdims=True)
    a = s + b_ref[0, 0]
    logits_ref[:] = a
    n = part.shape[0] // 2
    l0 = a[:n]
    l1 = a[n:]
    m = jnp.maximum(l0, l1)
    logz = m + jnp.log(jnp.exp(l0 - m) + jnp.exp(l1 - m))
    ll = jnp.where(lab_ref[:] == 0, l0, l1)
    loss_ref[0, 0] = jnp.mean(logz - ll)


def kernel(warrant0s, warrant1s, label_ids, emb, W, b):
    B = warrant0s.shape[0]
    bags = 2 * B
    iters = bags // NW // 2

    idx = jnp.concatenate(
        [warrant0s.astype(jnp.int32), warrant1s.astype(jnp.int32)],
        axis=0).reshape(B, 2 * L)

    wck = jnp.pad(W.reshape(-1), (0, NCHUNK * 16 - D)).reshape(NCHUNK, 16)

    emb16 = emb.reshape(V16, 16)

    part = _make_sc_call(bags, iters)(idx, emb16, wck)

    logits_col, loss_arr = pl.pallas_call(
        _tc_body,
        out_shape=[
            jax.ShapeDtypeStruct((bags, 1), jnp.float32),
            jax.ShapeDtypeStruct((1, 1), jnp.float32),
        ],
        in_specs=[
            pl.BlockSpec(memory_space=pltpu.VMEM),
            pl.BlockSpec(memory_space=pltpu.VMEM),
            pl.BlockSpec(memory_space=pltpu.SMEM),
        ],
        out_specs=[
            pl.BlockSpec(memory_space=pltpu.VMEM),
            pl.BlockSpec(memory_space=pltpu.SMEM),
        ],
    )(part, label_ids.astype(jnp.int32).reshape(B, 1), b.reshape(1, 1))

    logits = jnp.concatenate([logits_col[:B], logits_col[B:]], axis=1)
    return (loss_arr[0, 0], logits)


# speedup vs baseline: 1.6726x; 1.6726x over previous
import functools

import jax
import jax.numpy as jnp
from jax import lax
from jax.experimental import pallas as pl
from jax.experimental.pallas import tpu as pltpu
from jax.experimental.pallas import tpu_sc as plsc

D = 300
L = 50
NC, NS = 2, 16
NW = NC * NS
DP = 304
NCHUNK = 19
CHUNK_OFFS = tuple(c * 16 for c in range(NCHUNK))


def _sc_body(idx_hbm, emb_hbm, wck_hbm, out_hbm, idx_v, wck_v, out_v,
             rows_a, rows_b, sem_a, sem_b):
    wid = lax.axis_index("c") * NS + lax.axis_index("s")
    iters = idx_v.shape[0]
    row0 = wid * iters

    pltpu.sync_copy(idx_hbm.at[pl.ds(row0, iters)], idx_v)
    pltpu.sync_copy(wck_hbm, wck_v)

    def start(it, buf, sem):
        pltpu.async_copy(emb_hbm.at[idx_v.at[it]], buf, sem)

    def wait(it, buf, sem):
        pltpu.make_async_copy(emb_hbm.at[idx_v.at[it]], buf, sem).wait()

    def compute(rows, it):
        for half in range(2):
            base = half * L

            def body(r, accs):
                return tuple(
                    jnp.maximum(a, rows[base + r, pl.ds(off, 16)])
                    for a, off in zip(accs, CHUNK_OFFS))

            accs = tuple(rows[base, pl.ds(off, 16)] for off in CHUNK_OFFS)
            accs = lax.fori_loop(1, L, body, accs)
            psum = jnp.zeros((16,), jnp.float32)
            for c in range(NCHUNK):
                psum = psum + accs[c] * wck_v[c, :]
            out_v[2 * it + half, :] = psum

    start(0, rows_a, sem_a)

    def step2(k2, _):
        it0 = 2 * k2
        it1 = 2 * k2 + 1
        start(it1, rows_b, sem_b)
        wait(it0, rows_a, sem_a)
        compute(rows_a, it0)
        it2 = jnp.minimum(it0 + 2, iters - 1)
        start(it2, rows_a, sem_a)
        wait(it1, rows_b, sem_b)
        compute(rows_b, it1)
        return _

    lax.fori_loop(0, iters // 2, step2, 0)
    wait(iters - 1, rows_a, sem_a)
    pltpu.sync_copy(out_v, out_hbm.at[pl.ds(row0 * 2, 2 * iters)])


def _make_sc_call(bags, iters):
    mesh = plsc.VectorSubcoreMesh(
        core_axis_name="c", subcore_axis_name="s",
        num_cores=NC, num_subcores=NS)
    return functools.partial(
        pl.kernel, mesh=mesh,
        compiler_params=pltpu.CompilerParams(
            use_tc_tiling_on_sc=False, needs_layout_passes=False),
        out_type=jax.ShapeDtypeStruct((bags, 16), jnp.float32),
        scratch_types=[
            pltpu.VMEM((iters, 100), jnp.int32),
            pltpu.VMEM((NCHUNK, 16), jnp.float32),
            pltpu.VMEM((2 * iters, 16), jnp.float32),
            pltpu.VMEM((2 * L, DP), jnp.float32),
            pltpu.VMEM((2 * L, DP), jnp.float32),
            pltpu.SemaphoreType.DMA,
            pltpu.SemaphoreType.DMA,
        ])(_sc_body)


def _pad_body(in_ref, out_ref):
    out_ref[:] = jnp.pad(in_ref[:], ((0, 0), (0, DP - D)))


def _pad_table(emb):
    V = emb.shape[0]
    rows = 2000
    return pl.pallas_call(
        _pad_body,
        grid=(V // rows,),
        in_specs=[pl.BlockSpec((rows, D), lambda i: (i, 0))],
        out_specs=pl.BlockSpec((rows, DP), lambda i: (i, 0)),
        out_shape=jax.ShapeDtypeStruct((V, DP), jnp.float32),
    )(emb)


def _tc_body(part_ref, lab_ref, b_ref, logits_ref, loss_ref):
    part = part_ref[:]
    s = jnp.sum(part, axis=1, keepdims=True)
    a = s + b_ref[0, 0]
    logits_ref[:] = a
    n = part.shape[0] // 2
    l0 = a[:n]
    l1 = a[n:]
    m = jnp.maximum(l0, l1)
    logz = m + jnp.log(jnp.exp(l0 - m) + jnp.exp(l1 - m))
    ll = jnp.where(lab_ref[:] == 0, l0, l1)
    loss_ref[0, 0] = jnp.mean(logz - ll)


def kernel(warrant0s, warrant1s, label_ids, emb, W, b):
    B = warrant0s.shape[0]
    bags = 2 * B
    iters = bags // NW // 2

    idx = jnp.concatenate(
        [warrant0s.astype(jnp.int32), warrant1s.astype(jnp.int32)],
        axis=0).reshape(B, 2 * L)

    wck = jnp.pad(W.reshape(-1), (0, DP - D)).reshape(NCHUNK, 16)

    emb_p = _pad_table(emb)

    part = _make_sc_call(bags, iters)(idx, emb_p, wck)

    logits_col, loss_arr = pl.pallas_call(
        _tc_body,
        out_shape=[
            jax.ShapeDtypeStruct((bags, 1), jnp.float32),
            jax.ShapeDtypeStruct((1, 1), jnp.float32),
        ],
        in_specs=[
            pl.BlockSpec(memory_space=pltpu.VMEM),
            pl.BlockSpec(memory_space=pltpu.VMEM),
            pl.BlockSpec(memory_space=pltpu.SMEM),
        ],
        out_specs=[
            pl.BlockSpec(memory_space=pltpu.VMEM),
            pl.BlockSpec(memory_space=pltpu.SMEM),
        ],
    )(part, label_ids.astype(jnp.int32).reshape(B, 1), b.reshape(1, 1))

    logits = jnp.concatenate([logits_col[:B], logits_col[B:]], axis=1)
    return (loss_arr[0, 0], logits)
